# baseline (speedup 1.0000x reference)
import math

import jax
import jax.numpy as jnp
from jax.experimental import pallas as pl
from jax.experimental.pallas import tpu as pltpu

G = 512
A = 32
N = G * A
H = 128
F = 64
R = 12
CUT = 5.0
NAF = 13
NMF = 8
OUT = 256
NTYPES = 101
EMB = 5

GB = 8
M = GB * A
PH = M * A // 2
AH = A // 2

_INTERPRET = False

_C0 = math.sqrt(2.0 / CUT)


def _block_kernel(x_ref, pos_ref, posc_ref, T_ref, Wn_ref, bn_ref,
                  Wh0_ref, Wr0_ref, Wuh0_ref, Wua0_ref, bu0_ref,
                  Wh1_ref, Wr1_ref, Wuh1_ref, Wua1_ref, bu1_ref,
                  Wh2_ref, Wr2_ref, Wuh2_ref, Wua2_ref, bu2_ref,
                  Wmol_ref, bmol_ref, W1g_ref, W1m_ref, bf1_ref,
                  W2_ref, bf2_ref, Wo_ref, out_ref):
    gelu = jax.nn.gelu
    f32 = jnp.float32
    i32 = jnp.int32

    xb = x_ref[...]
    posb = pos_ref[...]
    poscb = posc_ref[...]

    row = jax.lax.broadcasted_iota(i32, (M, 1), 0)
    first = (row % A == 0).astype(f32)
    molx = jnp.sum((xb * first).reshape(GB, A, NAF), axis=1)
    mol = jnp.dot(molx[:, NAF - NMF:], Wmol_ref[...],
                  preferred_element_type=f32) + bmol_ref[...]

    atype = jnp.clip((xb[:, 0:1] * NTYPES).astype(i32), 0, NTYPES - 1)
    lanes = jax.lax.broadcasted_iota(i32, (M, 128), 1)
    onehot = (lanes == atype).astype(f32)
    h = gelu(jnp.dot(onehot, T_ref[...], preferred_element_type=f32)
             + jnp.dot(xb[:, 1:], Wn_ref[...], preferred_element_type=f32)
             + bn_ref[...])

    posb6 = jnp.concatenate([posb, posb], axis=1)
    prow = jnp.broadcast_to(posb6.reshape(M, 1, 6),
                            (M, AH, 6)).reshape(PH, 6)
    pcol = jnp.broadcast_to(poscb.reshape(GB, 1, AH, 6),
                            (GB, A, AH, 6)).reshape(PH, 6)
    df = prow - pcol
    sq = df * df
    rep0 = jax.lax.broadcasted_iota(i32, (6, 2 * R), 0)
    rep1 = jax.lax.broadcasted_iota(i32, (6, 2 * R), 1)
    REP = (rep0 // 3 == rep1 // R).astype(f32)
    d2rep = jnp.dot(sq, REP, preferred_element_type=f32)
    drep = jnp.sqrt(d2rep + 1e-12)

    ridx = jax.lax.broadcasted_iota(i32, (PH, 1), 0)
    jp = ridx % AH
    ii = (ridx // AH) % A
    d_e = jnp.sqrt(sq[:, 0:1] + sq[:, 1:2] + sq[:, 2:3] + 1e-12)
    d_o = jnp.sqrt(sq[:, 3:4] + sq[:, 4:5] + sq[:, 5:6] + 1e-12)
    amp_e = jnp.where((d_e < CUT) & (ii != 2 * jp), _C0 / d_e, 0.0)
    amp_o = jnp.where((d_o < CUT) & (ii != 2 * jp + 1), _C0 / d_o, 0.0)
    acol = jnp.concatenate([amp_e, amp_o], axis=1)
    r2a = jax.lax.broadcasted_iota(i32, (2, 128), 0)
    r2b = jax.lax.broadcasted_iota(i32, (2, 128), 1)
    REP2 = (r2a == r2b // F).astype(f32)
    ampR = jnp.dot(acol, REP2, preferred_element_type=f32)
    penR = jnp.where(ampR > 0.0, 0.0, -200.0)

    kf2 = ((jax.lax.broadcasted_iota(i32, (1, 2 * R), 1) % R + 1)
           .astype(f32) * (math.pi / CUT))
    theta = drep * kf2
    n = jnp.round(theta * (0.5 / math.pi))
    v = theta - n * (2.0 * math.pi)
    v2 = v * v
    s = v * (0.9999994441442891 + v2 * (-0.1666651950620369 + v2 * (
        0.00833220729172304 + v2 * (-0.00019803942981621122 + v2 * (
            2.694818791282763e-06 + v2 * -2.0177080094133367e-08)))))

    se0 = jax.lax.broadcasted_iota(i32, (M // 2, M), 0)
    se1 = jax.lax.broadcasted_iota(i32, (M // 2, M), 1)
    SELE = (2 * se0 == se1).astype(f32)
    SELO = (2 * se0 + 1 == se1).astype(f32)

    for (Wh_ref, Wr_ref, Wuh_ref, Wua_ref, bu_ref) in (
            (Wh0_ref, Wr0_ref, Wuh0_ref, Wua0_ref, bu0_ref),
            (Wh1_ref, Wr1_ref, Wuh1_ref, Wua1_ref, bu1_ref),
            (Wh2_ref, Wr2_ref, Wuh2_ref, Wua2_ref, bu2_ref)):
        q = jnp.dot(h, Wh_ref[...], preferred_element_type=f32)
        q2 = jnp.concatenate(
            [jnp.dot(SELE, q, preferred_element_type=f32),
             jnp.dot(SELO, q, preferred_element_type=f32)], axis=1)
        qt = jnp.broadcast_to(q2.reshape(GB, 1, AH, 2 * F),
                              (GB, A, AH, 2 * F)).reshape(PH, 2 * F)
        z2 = jnp.dot(s, Wr_ref[...], preferred_element_type=f32)
        m = gelu(qt + z2 * ampR + penR)
        sj = jnp.sum(m.reshape(M, AH, 2 * F), axis=1)
        agg = sj[:, :F] + sj[:, F:]
        upd = gelu(jnp.dot(h, Wuh_ref[...], preferred_element_type=f32)
                   + jnp.dot(agg, Wua_ref[...], preferred_element_type=f32)
                   + bu_ref[...])
        h = h + upd

    xg = jnp.sum(h.reshape(GB, A, H), axis=1) * (1.0 / A)
    z = gelu(jnp.dot(xg, W1g_ref[...], preferred_element_type=f32)
             + jnp.dot(mol, W1m_ref[...], preferred_element_type=f32)
             + bf1_ref[...])
    z = gelu(jnp.dot(z, W2_ref[...], preferred_element_type=f32) + bf2_ref[...])
    out_ref[...] = jnp.dot(z, Wo_ref[...], preferred_element_type=f32)


def kernel(x, pos, batch, ptr, aux_ind, num_graphs, atom_emb, W_node, b_node,
           Wh0, Wr0, Wu0, bu0, Wh1, Wr1, Wu1, bu1, Wh2, Wr2, Wu2, bu2,
           W_mol, b_mol, W_fc1, b_fc1, W_fc2, b_fc2, W_out):
    f32 = jnp.float32
    T = jnp.zeros((128, H), f32).at[:NTYPES].set(
        atom_emb @ W_node[:EMB])
    Wn = W_node[EMB:]
    posc = pos.reshape(G, A // 2, 6)

    def blockdiag2(W):
        Z = jnp.zeros((2 * R, 2 * F), f32)
        return Z.at[:R, :F].set(W).at[R:, F:].set(W)

    row_specs = [
        pl.BlockSpec((M, NAF), lambda g: (g, 0)),
        pl.BlockSpec((M, 3), lambda g: (g, 0)),
        pl.BlockSpec((GB, A // 2, 6), lambda g: (g, 0, 0)),
    ]

    full = lambda a: pl.BlockSpec(a.shape, lambda g: tuple(0 for _ in a.shape))
    weights = [T, Wn, b_node.reshape(1, H),
               Wh0, blockdiag2(Wr0), Wu0[:H], Wu0[H:], bu0.reshape(1, H),
               Wh1, blockdiag2(Wr1), Wu1[:H], Wu1[H:], bu1.reshape(1, H),
               Wh2, blockdiag2(Wr2), Wu2[:H], Wu2[H:], bu2.reshape(1, H),
               W_mol, b_mol.reshape(1, NMF),
               W_fc1[:H], W_fc1[H:], b_fc1.reshape(1, H),
               W_fc2, b_fc2.reshape(1, H), W_out]

    out = pl.pallas_call(
        _block_kernel,
        grid=(G // GB,),
        in_specs=row_specs + [full(w) for w in weights],
        out_specs=pl.BlockSpec((GB, OUT), lambda g: (g, 0)),
        out_shape=jax.ShapeDtypeStruct((G, OUT), f32),
        compiler_params=pltpu.CompilerParams(
            dimension_semantics=("arbitrary",)),
        interpret=_INTERPRET,
    )(x, pos, posc, *weights)
    return out


# speedup vs baseline: 1.0755x; 1.0755x over previous
import math

import jax
import jax.numpy as jnp
from jax.experimental import pallas as pl
from jax.experimental.pallas import tpu as pltpu

G = 512
A = 32
N = G * A
H = 128
F = 64
R = 12
CUT = 5.0
NAF = 13
NMF = 8
OUT = 256
NTYPES = 101
EMB = 5

GB = 8
M = GB * A
PH = M * A // 2
AH = A // 2

_INTERPRET = False

_C0 = math.sqrt(2.0 / CUT)


def _block_kernel(x_ref, posr_ref, posc_ref, T_ref, Wn_ref, bn_ref,
                  Wh0_ref, Wr0_ref, Wuh0_ref, Wua0_ref, bu0_ref,
                  Wh1_ref, Wr1_ref, Wuh1_ref, Wua1_ref, bu1_ref,
                  Wh2_ref, Wr2_ref, Wuh2_ref, Wua2_ref, bu2_ref,
                  Wmol_ref, bmol_ref, W1g_ref, W1m_ref, bf1_ref,
                  W2_ref, bf2_ref, Wo_ref, out_ref):
    gelu = jax.nn.gelu
    f32 = jnp.float32
    i32 = jnp.int32

    xb = x_ref[...]
    poscb = posc_ref[...]

    row = jax.lax.broadcasted_iota(i32, (M, 1), 0)
    first = (row % A == 0).astype(f32)
    molx = jnp.sum((xb * first).reshape(GB, A, NAF), axis=1)
    mol = jnp.dot(molx[:, NAF - NMF:], Wmol_ref[...],
                  preferred_element_type=f32) + bmol_ref[...]

    atype = jnp.clip((xb[:, 0:1] * NTYPES).astype(i32), 0, NTYPES - 1)
    lanes = jax.lax.broadcasted_iota(i32, (M, 128), 1)
    onehot = (lanes == atype).astype(f32)
    h = gelu(jnp.dot(onehot, T_ref[...], preferred_element_type=f32)
             + jnp.dot(xb[:, 1:], Wn_ref[...], preferred_element_type=f32)
             + bn_ref[...])

    prow = jnp.broadcast_to(posr_ref[...].reshape(M, 1, 6),
                            (M, AH, 6)).reshape(PH, 6)
    pcol = jnp.broadcast_to(poscb.reshape(GB, 1, AH, 6),
                            (GB, A, AH, 6)).reshape(PH, 6)
    df = prow - pcol
    sq = df * df

    ridx = jax.lax.broadcasted_iota(i32, (PH, 1), 0)
    jp = ridx % AH
    ii = (ridx // AH) % A
    dcol = jnp.sqrt(sq[:, 0:2] + sq[:, 2:4] + sq[:, 4:6] + 1e-12)
    jj2 = 2 * jp + jax.lax.broadcasted_iota(i32, (PH, 2), 1)
    acol = jnp.where((dcol < CUT) & (ii != jj2), _C0 / dcol, 0.0)
    rda = jax.lax.broadcasted_iota(i32, (2, 2 * R), 0)
    rdb = jax.lax.broadcasted_iota(i32, (2, 2 * R), 1)
    REPD = (rda == rdb // R).astype(f32)
    drep = jnp.dot(dcol, REPD, preferred_element_type=f32)
    r2a = jax.lax.broadcasted_iota(i32, (2, 128), 0)
    r2b = jax.lax.broadcasted_iota(i32, (2, 128), 1)
    REP2 = (r2a == r2b // F).astype(f32)
    ampR = jnp.dot(acol, REP2, preferred_element_type=f32)
    penR = jnp.where(ampR > 0.0, 0.0, -200.0)

    kf2 = ((jax.lax.broadcasted_iota(i32, (1, 2 * R), 1) % R + 1)
           .astype(f32) * (math.pi / CUT))
    theta = drep * kf2
    n = jnp.round(theta * (0.5 / math.pi))
    v = theta - n * (2.0 * math.pi)
    v2 = v * v
    s = v * (0.9999994441442891 + v2 * (-0.1666651950620369 + v2 * (
        0.00833220729172304 + v2 * (-0.00019803942981621122 + v2 * (
            2.694818791282763e-06 + v2 * -2.0177080094133367e-08)))))

    se0 = jax.lax.broadcasted_iota(i32, (M // 2, M), 0)
    se1 = jax.lax.broadcasted_iota(i32, (M // 2, M), 1)
    SELE = (2 * se0 == se1).astype(f32)
    SELO = (2 * se0 + 1 == se1).astype(f32)

    for (Wh_ref, Wr_ref, Wuh_ref, Wua_ref, bu_ref) in (
            (Wh0_ref, Wr0_ref, Wuh0_ref, Wua0_ref, bu0_ref),
            (Wh1_ref, Wr1_ref, Wuh1_ref, Wua1_ref, bu1_ref),
            (Wh2_ref, Wr2_ref, Wuh2_ref, Wua2_ref, bu2_ref)):
        q = jnp.dot(h, Wh_ref[...], preferred_element_type=f32)
        q2 = jnp.concatenate(
            [jnp.dot(SELE, q, preferred_element_type=f32),
             jnp.dot(SELO, q, preferred_element_type=f32)], axis=1)
        qt = jnp.broadcast_to(q2.reshape(GB, 1, AH, 2 * F),
                              (GB, A, AH, 2 * F)).reshape(PH, 2 * F)
        z2 = jnp.dot(s, Wr_ref[...], preferred_element_type=f32)
        m = gelu(qt + z2 * ampR + penR)
        sj = jnp.sum(m.reshape(M, AH, 2 * F), axis=1)
        agg = sj[:, :F] + sj[:, F:]
        upd = gelu(jnp.dot(h, Wuh_ref[...], preferred_element_type=f32)
                   + jnp.dot(agg, Wua_ref[...], preferred_element_type=f32)
                   + bu_ref[...])
        h = h + upd

    xg = jnp.sum(h.reshape(GB, A, H), axis=1) * (1.0 / A)
    z = gelu(jnp.dot(xg, W1g_ref[...], preferred_element_type=f32)
             + jnp.dot(mol, W1m_ref[...], preferred_element_type=f32)
             + bf1_ref[...])
    z = gelu(jnp.dot(z, W2_ref[...], preferred_element_type=f32) + bf2_ref[...])
    out_ref[...] = jnp.dot(z, Wo_ref[...], preferred_element_type=f32)


def kernel(x, pos, batch, ptr, aux_ind, num_graphs, atom_emb, W_node, b_node,
           Wh0, Wr0, Wu0, bu0, Wh1, Wr1, Wu1, bu1, Wh2, Wr2, Wu2, bu2,
           W_mol, b_mol, W_fc1, b_fc1, W_fc2, b_fc2, W_out):
    f32 = jnp.float32
    T = jnp.zeros((128, H), f32).at[:NTYPES].set(
        atom_emb @ W_node[:EMB])
    Wn = W_node[EMB:]
    posr = jnp.repeat(pos, 2, axis=1)
    posc = pos.reshape(G, A // 2, 2, 3).transpose(0, 1, 3, 2).reshape(
        G, A // 2, 6)

    def blockdiag2(W):
        Z = jnp.zeros((2 * R, 2 * F), f32)
        return Z.at[:R, :F].set(W).at[R:, F:].set(W)

    row_specs = [
        pl.BlockSpec((M, NAF), lambda g: (g, 0)),
        pl.BlockSpec((M, 6), lambda g: (g, 0)),
        pl.BlockSpec((GB, A // 2, 6), lambda g: (g, 0, 0)),
    ]

    full = lambda a: pl.BlockSpec(a.shape, lambda g: tuple(0 for _ in a.shape))
    weights = [T, Wn, b_node.reshape(1, H),
               Wh0, blockdiag2(Wr0), Wu0[:H], Wu0[H:], bu0.reshape(1, H),
               Wh1, blockdiag2(Wr1), Wu1[:H], Wu1[H:], bu1.reshape(1, H),
               Wh2, blockdiag2(Wr2), Wu2[:H], Wu2[H:], bu2.reshape(1, H),
               W_mol, b_mol.reshape(1, NMF),
               W_fc1[:H], W_fc1[H:], b_fc1.reshape(1, H),
               W_fc2, b_fc2.reshape(1, H), W_out]

    out = pl.pallas_call(
        _block_kernel,
        grid=(G // GB,),
        in_specs=row_specs + [full(w) for w in weights],
        out_specs=pl.BlockSpec((GB, OUT), lambda g: (g, 0)),
        out_shape=jax.ShapeDtypeStruct((G, OUT), f32),
        compiler_params=pltpu.CompilerParams(
            dimension_semantics=("arbitrary",)),
        interpret=_INTERPRET,
    )(x, posr, posc, *weights)
    return out


# speedup vs baseline: 1.3176x; 1.2251x over previous
import math

import jax
import jax.numpy as jnp
from jax.experimental import pallas as pl
from jax.experimental.pallas import tpu as pltpu

G = 512
A = 32
N = G * A
H = 128
F = 64
R = 12
CUT = 5.0
NAF = 13
NMF = 8
OUT = 256
NTYPES = 101
EMB = 5

GB = 8
M = GB * A
P8 = 8
AQ = A // P8
PQ = M * AQ
WL = P8 * F

_INTERPRET = False

_C0 = math.sqrt(2.0 / CUT)


def _block_kernel(x_ref, posr_ref, posc_ref, T_ref, Wn_ref, bn_ref,
                  Wh0_ref, Wr0_ref, Wuh0_ref, Wua0_ref, bu0_ref,
                  Wh1_ref, Wr1_ref, Wuh1_ref, Wua1_ref, bu1_ref,
                  Wh2_ref, Wr2_ref, Wuh2_ref, Wua2_ref, bu2_ref,
                  Wmol_ref, bmol_ref, W1g_ref, W1m_ref, bf1_ref,
                  W2_ref, bf2_ref, Wo_ref, out_ref):
    gelu = jax.nn.gelu
    f32 = jnp.float32
    i32 = jnp.int32

    xb = x_ref[...]
    poscb = posc_ref[...]

    row = jax.lax.broadcasted_iota(i32, (M, 1), 0)
    first = (row % A == 0).astype(f32)
    molx = jnp.sum((xb * first).reshape(GB, A, NAF), axis=1)
    mol = jnp.dot(molx[:, NAF - NMF:], Wmol_ref[...],
                  preferred_element_type=f32) + bmol_ref[...]

    atype = jnp.clip((xb[:, 0:1] * NTYPES).astype(i32), 0, NTYPES - 1)
    lanes = jax.lax.broadcasted_iota(i32, (M, 128), 1)
    onehot = (lanes == atype).astype(f32)
    h = gelu(jnp.dot(onehot, T_ref[...], preferred_element_type=f32)
             + jnp.dot(xb[:, 1:], Wn_ref[...], preferred_element_type=f32)
             + bn_ref[...])

    prow = jnp.broadcast_to(posr_ref[...].reshape(M, 1, 3 * P8),
                            (M, AQ, 3 * P8)).reshape(PQ, 3 * P8)
    pcol = jnp.broadcast_to(poscb.reshape(GB, 1, AQ, 3 * P8),
                            (GB, A, AQ, 3 * P8)).reshape(PQ, 3 * P8)
    df = prow - pcol
    sq = df * df

    ridx = jax.lax.broadcasted_iota(i32, (PQ, 1), 0)
    jo = ridx % AQ
    ii = (ridx // AQ) % A
    jj = P8 * jo + jax.lax.broadcasted_iota(i32, (PQ, P8), 1)
    dcol = jnp.sqrt(sq[:, 0:P8] + sq[:, P8:2 * P8] + sq[:, 2 * P8:] + 1e-12)
    adj = (dcol < CUT) & (ii != jj)
    acol = jnp.where(adj, _C0 / dcol, 0.0)
    pencol = jnp.where(adj, 0.0, -200.0)

    ra = jax.lax.broadcasted_iota(i32, (P8, P8 * R), 0)
    rb = jax.lax.broadcasted_iota(i32, (P8, P8 * R), 1)
    REPR = (ra == rb // R).astype(f32)
    r2a = jax.lax.broadcasted_iota(i32, (P8, WL), 0)
    r2b = jax.lax.broadcasted_iota(i32, (P8, WL), 1)
    REPW = (r2a == r2b // F).astype(f32)
    drep = jnp.dot(dcol, REPR, preferred_element_type=f32)
    penR = jnp.dot(pencol, REPW, preferred_element_type=f32)

    kf = ((jax.lax.broadcasted_iota(i32, (1, P8 * R), 1) % R + 1)
          .astype(f32) * (math.pi / CUT))
    theta = drep * kf
    n = jnp.round(theta * (0.5 / math.pi))
    v = theta - n * (2.0 * math.pi)
    v2 = v * v
    s = v * (0.9999994441442891 + v2 * (-0.1666651950620369 + v2 * (
        0.00833220729172304 + v2 * (-0.00019803942981621122 + v2 * (
            2.694818791282763e-06 + v2 * -2.0177080094133367e-08)))))
    samp = s * jnp.dot(acol, REPR, preferred_element_type=f32)

    se0 = jax.lax.broadcasted_iota(i32, (M // P8, M), 0)
    se1 = jax.lax.broadcasted_iota(i32, (M // P8, M), 1)
    SELS = [(P8 * se0 + p == se1).astype(f32) for p in range(P8)]

    for (Wh_ref, Wr_ref, Wuh_ref, Wua_ref, bu_ref) in (
            (Wh0_ref, Wr0_ref, Wuh0_ref, Wua0_ref, bu0_ref),
            (Wh1_ref, Wr1_ref, Wuh1_ref, Wua1_ref, bu1_ref),
            (Wh2_ref, Wr2_ref, Wuh2_ref, Wua2_ref, bu2_ref)):
        q = jnp.dot(h, Wh_ref[...], preferred_element_type=f32)
        q8 = jnp.concatenate(
            [jnp.dot(S, q, preferred_element_type=f32) for S in SELS],
            axis=1)
        qt = jnp.broadcast_to(q8.reshape(GB, 1, AQ, WL),
                              (GB, A, AQ, WL)).reshape(PQ, WL)
        z2 = jnp.dot(samp, Wr_ref[...], preferred_element_type=f32)
        m = gelu(qt + z2 + penR)
        mw = jnp.dot(m, Wua_ref[...], preferred_element_type=f32)
        aggw = jnp.sum(mw.reshape(M, AQ, H), axis=1)
        upd = gelu(jnp.dot(h, Wuh_ref[...], preferred_element_type=f32)
                   + aggw + bu_ref[...])
        h = h + upd

    xg = jnp.sum(h.reshape(GB, A, H), axis=1) * (1.0 / A)
    z = gelu(jnp.dot(xg, W1g_ref[...], preferred_element_type=f32)
             + jnp.dot(mol, W1m_ref[...], preferred_element_type=f32)
             + bf1_ref[...])
    z = gelu(jnp.dot(z, W2_ref[...], preferred_element_type=f32) + bf2_ref[...])
    out_ref[...] = jnp.dot(z, Wo_ref[...], preferred_element_type=f32)


def kernel(x, pos, batch, ptr, aux_ind, num_graphs, atom_emb, W_node, b_node,
           Wh0, Wr0, Wu0, bu0, Wh1, Wr1, Wu1, bu1, Wh2, Wr2, Wu2, bu2,
           W_mol, b_mol, W_fc1, b_fc1, W_fc2, b_fc2, W_out):
    f32 = jnp.float32
    T = jnp.zeros((128, H), f32).at[:NTYPES].set(
        atom_emb @ W_node[:EMB])
    Wn = W_node[EMB:]
    posr = jnp.repeat(pos, P8, axis=1)
    posc = pos.reshape(G, AQ, P8, 3).transpose(0, 1, 3, 2).reshape(
        G, AQ, 3 * P8)

    def blockdiag8(W):
        Z = jnp.zeros((P8 * R, WL), f32)
        for p in range(P8):
            Z = Z.at[p * R:(p + 1) * R, p * F:(p + 1) * F].set(W)
        return Z

    row_specs = [
        pl.BlockSpec((M, NAF), lambda g: (g, 0)),
        pl.BlockSpec((M, 3 * P8), lambda g: (g, 0)),
        pl.BlockSpec((GB, AQ, 3 * P8), lambda g: (g, 0, 0)),
    ]

    full = lambda a: pl.BlockSpec(a.shape, lambda g: tuple(0 for _ in a.shape))
    tile8 = lambda Wua: jnp.tile(Wua, (P8, 1))
    weights = [T, Wn, b_node.reshape(1, H),
               Wh0, blockdiag8(Wr0), Wu0[:H], tile8(Wu0[H:]), bu0.reshape(1, H),
               Wh1, blockdiag8(Wr1), Wu1[:H], tile8(Wu1[H:]), bu1.reshape(1, H),
               Wh2, blockdiag8(Wr2), Wu2[:H], tile8(Wu2[H:]), bu2.reshape(1, H),
               W_mol, b_mol.reshape(1, NMF),
               W_fc1[:H], W_fc1[H:], b_fc1.reshape(1, H),
               W_fc2, b_fc2.reshape(1, H), W_out]

    out = pl.pallas_call(
        _block_kernel,
        grid=(G // GB,),
        in_specs=row_specs + [full(w) for w in weights],
        out_specs=pl.BlockSpec((GB, OUT), lambda g: (g, 0)),
        out_shape=jax.ShapeDtypeStruct((G, OUT), f32),
        compiler_params=pltpu.CompilerParams(
            dimension_semantics=("arbitrary",)),
        interpret=_INTERPRET,
    )(x, posr, posc, *weights)
    return out


# speedup vs baseline: 1.5566x; 1.1814x over previous
import math

import jax
import jax.numpy as jnp
from jax.experimental import pallas as pl
from jax.experimental.pallas import tpu as pltpu

G = 512
A = 32
N = G * A
H = 128
F = 64
R = 12
CUT = 5.0
NAF = 13
NMF = 8
OUT = 256
NTYPES = 101
EMB = 5

GB = 8
M = GB * A
P8 = 8
AQ = A // P8
PQ = M * AQ
WL = P8 * F

_INTERPRET = False

_C0 = math.sqrt(2.0 / CUT)


def _block_kernel(x_ref, posr_ref, posc_ref, T_ref, Wn_ref, bn_ref,
                  Wh0_ref, Wr0_ref, Wuh0_ref, Wua0_ref, bu0_ref,
                  Wh1_ref, Wr1_ref, Wuh1_ref, Wua1_ref, bu1_ref,
                  Wh2_ref, Wr2_ref, Wuh2_ref, Wua2_ref, bu2_ref,
                  Wmol_ref, bmol_ref, W1g_ref, W1m_ref, bf1_ref,
                  W2_ref, bf2_ref, Wo_ref, out_ref):
    gelu = jax.nn.gelu
    f32 = jnp.float32
    i32 = jnp.int32

    xb = x_ref[...]
    poscb = posc_ref[...]

    row = jax.lax.broadcasted_iota(i32, (M, 1), 0)
    first = (row % A == 0).astype(f32)
    molx = jnp.sum((xb * first).reshape(GB, A, NAF), axis=1)
    mol = jnp.dot(molx[:, NAF - NMF:], Wmol_ref[...],
                  preferred_element_type=f32) + bmol_ref[...]

    atype = jnp.clip((xb[:, 0:1] * NTYPES).astype(i32), 0, NTYPES - 1)
    lanes = jax.lax.broadcasted_iota(i32, (M, 128), 1)
    onehot = (lanes == atype).astype(f32)
    h = gelu(jnp.dot(onehot, T_ref[...], preferred_element_type=f32)
             + jnp.dot(xb[:, 1:], Wn_ref[...], preferred_element_type=f32)
             + bn_ref[...])

    prow = jnp.broadcast_to(posr_ref[...].reshape(1, M, 3 * P8),
                            (AQ, M, 3 * P8)).reshape(PQ, 3 * P8)
    pcol = jnp.broadcast_to(poscb.reshape(AQ, GB, 1, 3 * P8),
                            (AQ, GB, A, 3 * P8)).reshape(PQ, 3 * P8)
    df = prow - pcol
    sq = df * df

    ridx = jax.lax.broadcasted_iota(i32, (PQ, 1), 0)
    jo = ridx // M
    ii = ridx % A
    jj = P8 * jo + jax.lax.broadcasted_iota(i32, (PQ, P8), 1)
    dcol = jnp.sqrt(sq[:, 0:P8] + sq[:, P8:2 * P8] + sq[:, 2 * P8:] + 1e-12)
    adj = (dcol < CUT) & (ii != jj)
    acol = jnp.where(adj, _C0 / dcol, 0.0)
    pencol = jnp.where(adj, 0.0, -200.0)

    ra = jax.lax.broadcasted_iota(i32, (P8, P8 * R), 0)
    rb = jax.lax.broadcasted_iota(i32, (P8, P8 * R), 1)
    REPR = (ra == rb // R).astype(f32)
    drep = jnp.dot(dcol, REPR, preferred_element_type=f32)

    kf = ((jax.lax.broadcasted_iota(i32, (1, P8 * R), 1) % R + 1)
          .astype(f32) * (math.pi / CUT))
    theta = drep * kf
    n = jnp.round(theta * (0.5 / math.pi))
    v = theta - n * (2.0 * math.pi)
    v2 = v * v
    s = v * (0.9999994441442891 + v2 * (-0.1666651950620369 + v2 * (
        0.00833220729172304 + v2 * (-0.00019803942981621122 + v2 * (
            2.694818791282763e-06 + v2 * -2.0177080094133367e-08)))))
    samp = s * jnp.dot(acol, REPR, preferred_element_type=f32)
    saug = jnp.concatenate([samp, pencol], axis=1)

    se0 = jax.lax.broadcasted_iota(i32, (M // P8, M), 0)
    se1 = jax.lax.broadcasted_iota(i32, (M // P8, M), 1)
    SELS = [(A * (se0 % GB) + P8 * (se0 // GB) + p == se1).astype(f32)
            for p in range(P8)]

    for (Wh_ref, Wr_ref, Wuh_ref, Wua_ref, bu_ref) in (
            (Wh0_ref, Wr0_ref, Wuh0_ref, Wua0_ref, bu0_ref),
            (Wh1_ref, Wr1_ref, Wuh1_ref, Wua1_ref, bu1_ref),
            (Wh2_ref, Wr2_ref, Wuh2_ref, Wua2_ref, bu2_ref)):
        q = jnp.dot(h, Wh_ref[...], preferred_element_type=f32)
        q8 = jnp.concatenate(
            [jnp.dot(S, q, preferred_element_type=f32) for S in SELS],
            axis=1)
        qt = jnp.broadcast_to(q8.reshape(AQ, GB, 1, WL),
                              (AQ, GB, A, WL)).reshape(PQ, WL)
        z2 = jnp.dot(saug, Wr_ref[...], preferred_element_type=f32)
        m = gelu(qt + z2)
        mw = jnp.dot(m, Wua_ref[...], preferred_element_type=f32)
        aggw = jnp.sum(mw.reshape(AQ, M, H), axis=0)
        upd = gelu(jnp.dot(h, Wuh_ref[...], preferred_element_type=f32)
                   + aggw + bu_ref[...])
        h = h + upd

    xg = jnp.sum(h.reshape(GB, A, H), axis=1) * (1.0 / A)
    z = gelu(jnp.dot(xg, W1g_ref[...], preferred_element_type=f32)
             + jnp.dot(mol, W1m_ref[...], preferred_element_type=f32)
             + bf1_ref[...])
    z = gelu(jnp.dot(z, W2_ref[...], preferred_element_type=f32) + bf2_ref[...])
    out_ref[...] = jnp.dot(z, Wo_ref[...], preferred_element_type=f32)


def kernel(x, pos, batch, ptr, aux_ind, num_graphs, atom_emb, W_node, b_node,
           Wh0, Wr0, Wu0, bu0, Wh1, Wr1, Wu1, bu1, Wh2, Wr2, Wu2, bu2,
           W_mol, b_mol, W_fc1, b_fc1, W_fc2, b_fc2, W_out):
    f32 = jnp.float32
    T = jnp.zeros((128, H), f32).at[:NTYPES].set(
        atom_emb @ W_node[:EMB])
    Wn = W_node[EMB:]
    posr = jnp.repeat(pos, P8, axis=1)
    posc = pos.reshape(G // GB, GB, AQ, P8, 3).transpose(0, 2, 1, 4, 3).reshape(
        G // GB, AQ, GB, 3 * P8)

    def blockdiag8(W):
        Z = jnp.zeros((P8 * R + P8, WL), f32)
        for p in range(P8):
            Z = Z.at[p * R:(p + 1) * R, p * F:(p + 1) * F].set(W)
            Z = Z.at[P8 * R + p, p * F:(p + 1) * F].set(1.0)
        return Z

    row_specs = [
        pl.BlockSpec((M, NAF), lambda g: (g, 0)),
        pl.BlockSpec((M, 3 * P8), lambda g: (g, 0)),
        pl.BlockSpec((1, AQ, GB, 3 * P8), lambda g: (g, 0, 0, 0)),
    ]

    full = lambda a: pl.BlockSpec(a.shape, lambda g: tuple(0 for _ in a.shape))
    tile8 = lambda Wua: jnp.tile(Wua, (P8, 1))
    weights = [T, Wn, b_node.reshape(1, H),
               Wh0, blockdiag8(Wr0), Wu0[:H], tile8(Wu0[H:]), bu0.reshape(1, H),
               Wh1, blockdiag8(Wr1), Wu1[:H], tile8(Wu1[H:]), bu1.reshape(1, H),
               Wh2, blockdiag8(Wr2), Wu2[:H], tile8(Wu2[H:]), bu2.reshape(1, H),
               W_mol, b_mol.reshape(1, NMF),
               W_fc1[:H], W_fc1[H:], b_fc1.reshape(1, H),
               W_fc2, b_fc2.reshape(1, H), W_out]

    out = pl.pallas_call(
        _block_kernel,
        grid=(G // GB,),
        in_specs=row_specs + [full(w) for w in weights],
        out_specs=pl.BlockSpec((GB, OUT), lambda g: (g, 0)),
        out_shape=jax.ShapeDtypeStruct((G, OUT), f32),
        compiler_params=pltpu.CompilerParams(
            dimension_semantics=("arbitrary",)),
        interpret=_INTERPRET,
    )(x, posr, posc, *weights)
    return out


# speedup vs baseline: 1.5844x; 1.0179x over previous
import math

import jax
import jax.numpy as jnp
from jax.experimental import pallas as pl
from jax.experimental.pallas import tpu as pltpu

G = 512
A = 32
N = G * A
H = 128
F = 64
R = 12
CUT = 5.0
NAF = 13
NMF = 8
OUT = 256
NTYPES = 101
EMB = 5

GB = 8
M = GB * A
P8 = 8
AQ = A // P8
PQ = M * AQ
WL = P8 * F

_INTERPRET = False

_C0 = math.sqrt(2.0 / CUT)


def _block_kernel(x_ref, posr_ref, posc_ref, T_ref, Wn_ref, bn_ref,
                  Wh0_ref, Wr0_ref, Wuh0_ref, Wua0_ref, bu0_ref,
                  Wh1_ref, Wr1_ref, Wuh1_ref, Wua1_ref, bu1_ref,
                  Wh2_ref, Wr2_ref, Wuh2_ref, Wua2_ref, bu2_ref,
                  Wmol_ref, bmol_ref, W1g_ref, W1m_ref, bf1_ref,
                  W2_ref, bf2_ref, Wo_ref, out_ref):
    gelu = jax.nn.gelu
    f32 = jnp.float32
    i32 = jnp.int32

    xb = x_ref[...]
    poscb = posc_ref[...]

    row = jax.lax.broadcasted_iota(i32, (M, 1), 0)
    first = (row % A == 0).astype(f32)
    molx = jnp.sum((xb * first).reshape(GB, A, NAF), axis=1)
    mol = jnp.dot(molx[:, NAF - NMF:], Wmol_ref[...],
                  preferred_element_type=f32) + bmol_ref[...]

    atype = jnp.clip((xb[:, 0:1] * NTYPES).astype(i32), 0, NTYPES - 1)
    lanes = jax.lax.broadcasted_iota(i32, (M, 128), 1)
    onehot = (lanes == atype).astype(f32)
    h = gelu(jnp.dot(onehot, T_ref[...], preferred_element_type=f32)
             + jnp.dot(xb[:, 1:], Wn_ref[...], preferred_element_type=f32)
             + bn_ref[...])

    prow = jnp.broadcast_to(posr_ref[...].reshape(1, M, 3 * P8),
                            (AQ, M, 3 * P8)).reshape(PQ, 3 * P8)
    pcol = jnp.broadcast_to(poscb.reshape(AQ, GB, 1, 3 * P8),
                            (AQ, GB, A, 3 * P8)).reshape(PQ, 3 * P8)
    df = prow - pcol
    sq = df * df

    ridx = jax.lax.broadcasted_iota(i32, (PQ, 1), 0)
    jo = ridx // M
    ii = ridx % A
    jj = P8 * jo + jax.lax.broadcasted_iota(i32, (PQ, P8), 1)
    dcol = jnp.sqrt(sq[:, 0:P8] + sq[:, P8:2 * P8] + sq[:, 2 * P8:] + 1e-12)
    adj = (dcol < CUT) & (ii != jj)
    acol = jnp.where(adj, _C0 / dcol, 0.0)
    pencol = jnp.where(adj, 0.0, -200.0)

    ra = jax.lax.broadcasted_iota(i32, (P8, P8 * R), 0)
    rb = jax.lax.broadcasted_iota(i32, (P8, P8 * R), 1)
    REPR = (ra == rb // R).astype(f32)
    drep = jnp.dot(dcol, REPR, preferred_element_type=f32)

    kf = ((jax.lax.broadcasted_iota(i32, (1, P8 * R), 1) % R + 1)
          .astype(f32) * (math.pi / CUT))
    theta = drep * kf
    n = jnp.round(theta * (0.5 / math.pi))
    v = theta - n * (2.0 * math.pi)
    v2 = v * v
    s = v * (0.9999994441442891 + v2 * (-0.1666651950620369 + v2 * (
        0.00833220729172304 + v2 * (-0.00019803942981621122 + v2 * (
            2.694818791282763e-06 + v2 * -2.0177080094133367e-08)))))
    samp = s * jnp.dot(acol, REPR, preferred_element_type=f32)
    saug = jnp.concatenate([samp, pencol], axis=1)

    se0 = jax.lax.broadcasted_iota(i32, (M // P8, M), 0)
    se1 = jax.lax.broadcasted_iota(i32, (M // P8, M), 1)
    SELS = [(A * (se0 % GB) + P8 * (se0 // GB) + p == se1).astype(f32)
            for p in range(P8)]

    for (Wh_ref, Wr_ref, Wuh_ref, Wua_ref, bu_ref) in (
            (Wh0_ref, Wr0_ref, Wuh0_ref, Wua0_ref, bu0_ref),
            (Wh1_ref, Wr1_ref, Wuh1_ref, Wua1_ref, bu1_ref),
            (Wh2_ref, Wr2_ref, Wuh2_ref, Wua2_ref, bu2_ref)):
        q = jnp.dot(h, Wh_ref[...], preferred_element_type=f32)
        q8 = jnp.concatenate(
            [jnp.dot(S, q, preferred_element_type=f32) for S in SELS],
            axis=1)
        qt = jnp.broadcast_to(q8.reshape(AQ, GB, 1, WL),
                              (AQ, GB, A, WL)).reshape(PQ, WL)
        z2 = jnp.dot(saug, Wr_ref[...], preferred_element_type=f32)
        xm = qt + z2
        wm = xm * (0.7978845608028654 + 0.035677408136300125 * (xm * xm))
        rm = 0.5 * xm
        m = rm + rm * jnp.tanh(wm)
        mw = jnp.dot(m, Wua_ref[...], preferred_element_type=f32)
        aggw = jnp.sum(mw.reshape(AQ, M, H), axis=0)
        upd = gelu(jnp.dot(h, Wuh_ref[...], preferred_element_type=f32)
                   + aggw + bu_ref[...])
        h = h + upd

    xg = jnp.sum(h.reshape(GB, A, H), axis=1) * (1.0 / A)
    z = gelu(jnp.dot(xg, W1g_ref[...], preferred_element_type=f32)
             + jnp.dot(mol, W1m_ref[...], preferred_element_type=f32)
             + bf1_ref[...])
    z = gelu(jnp.dot(z, W2_ref[...], preferred_element_type=f32) + bf2_ref[...])
    out_ref[...] = jnp.dot(z, Wo_ref[...], preferred_element_type=f32)


def kernel(x, pos, batch, ptr, aux_ind, num_graphs, atom_emb, W_node, b_node,
           Wh0, Wr0, Wu0, bu0, Wh1, Wr1, Wu1, bu1, Wh2, Wr2, Wu2, bu2,
           W_mol, b_mol, W_fc1, b_fc1, W_fc2, b_fc2, W_out):
    f32 = jnp.float32
    T = jnp.zeros((128, H), f32).at[:NTYPES].set(
        atom_emb @ W_node[:EMB])
    Wn = W_node[EMB:]
    posr = jnp.repeat(pos, P8, axis=1)
    posc = pos.reshape(G // GB, GB, AQ, P8, 3).transpose(0, 2, 1, 4, 3).reshape(
        G // GB, AQ, GB, 3 * P8)

    def blockdiag8(W):
        Z = jnp.zeros((P8 * R + P8, WL), f32)
        for p in range(P8):
            Z = Z.at[p * R:(p + 1) * R, p * F:(p + 1) * F].set(W)
            Z = Z.at[P8 * R + p, p * F:(p + 1) * F].set(1.0)
        return Z

    row_specs = [
        pl.BlockSpec((M, NAF), lambda g: (g, 0)),
        pl.BlockSpec((M, 3 * P8), lambda g: (g, 0)),
        pl.BlockSpec((1, AQ, GB, 3 * P8), lambda g: (g, 0, 0, 0)),
    ]

    full = lambda a: pl.BlockSpec(a.shape, lambda g: tuple(0 for _ in a.shape))
    tile8 = lambda Wua: jnp.tile(Wua, (P8, 1))
    weights = [T, Wn, b_node.reshape(1, H),
               Wh0, blockdiag8(Wr0), Wu0[:H], tile8(Wu0[H:]), bu0.reshape(1, H),
               Wh1, blockdiag8(Wr1), Wu1[:H], tile8(Wu1[H:]), bu1.reshape(1, H),
               Wh2, blockdiag8(Wr2), Wu2[:H], tile8(Wu2[H:]), bu2.reshape(1, H),
               W_mol, b_mol.reshape(1, NMF),
               W_fc1[:H], W_fc1[H:], b_fc1.reshape(1, H),
               W_fc2, b_fc2.reshape(1, H), W_out]

    out = pl.pallas_call(
        _block_kernel,
        grid=(G // GB,),
        in_specs=row_specs + [full(w) for w in weights],
        out_specs=pl.BlockSpec((GB, OUT), lambda g: (g, 0)),
        out_shape=jax.ShapeDtypeStruct((G, OUT), f32),
        compiler_params=pltpu.CompilerParams(
            dimension_semantics=("parallel",)),
        interpret=_INTERPRET,
    )(x, posr, posc, *weights)
    return out


# speedup vs baseline: 1.8485x; 1.1667x over previous
import math

import jax
import jax.numpy as jnp
from jax.experimental import pallas as pl
from jax.experimental.pallas import tpu as pltpu

G = 512
A = 32
N = G * A
H = 128
F = 64
R = 12
CUT = 5.0
NAF = 13
NMF = 8
OUT = 256
NTYPES = 101
EMB = 5

GB = 16
M = GB * A
P8 = 8
AQ = A // P8
PQ = M * AQ
WL = P8 * F

_INTERPRET = False

_C0 = math.sqrt(2.0 / CUT)


def _block_kernel(x_ref, posr_ref, posc_ref, T_ref, Wn_ref, bn_ref,
                  Wh0_ref, Wr0_ref, Wuh0_ref, Wua0_ref, bu0_ref,
                  Wh1_ref, Wr1_ref, Wuh1_ref, Wua1_ref, bu1_ref,
                  Wh2_ref, Wr2_ref, Wuh2_ref, Wua2_ref, bu2_ref,
                  Wmol_ref, bmol_ref, W1g_ref, W1m_ref, bf1_ref,
                  W2_ref, bf2_ref, Wo_ref, out_ref):
    gelu = jax.nn.gelu
    f32 = jnp.float32
    i32 = jnp.int32

    xb = x_ref[...]
    poscb = posc_ref[...]

    row = jax.lax.broadcasted_iota(i32, (M, 1), 0)
    first = (row % A == 0).astype(f32)
    molx = jnp.sum((xb * first).reshape(GB, A, NAF), axis=1)
    mol = jnp.dot(molx[:, NAF - NMF:], Wmol_ref[...],
                  preferred_element_type=f32) + bmol_ref[...]

    atype = jnp.clip((xb[:, 0:1] * NTYPES).astype(i32), 0, NTYPES - 1)
    lanes = jax.lax.broadcasted_iota(i32, (M, 128), 1)
    onehot = (lanes == atype).astype(f32)
    h = gelu(jnp.dot(onehot, T_ref[...], preferred_element_type=f32)
             + jnp.dot(xb[:, 1:], Wn_ref[...], preferred_element_type=f32)
             + bn_ref[...])

    prow = jnp.broadcast_to(posr_ref[...].reshape(1, M, 3 * P8),
                            (AQ, M, 3 * P8)).reshape(PQ, 3 * P8)
    pcol = jnp.broadcast_to(poscb.reshape(AQ, GB, 1, 3 * P8),
                            (AQ, GB, A, 3 * P8)).reshape(PQ, 3 * P8)
    df = prow - pcol
    sq = df * df

    ridx = jax.lax.broadcasted_iota(i32, (PQ, 1), 0)
    jo = ridx // M
    ii = ridx % A
    jj = P8 * jo + jax.lax.broadcasted_iota(i32, (PQ, P8), 1)
    dcol = jnp.sqrt(sq[:, 0:P8] + sq[:, P8:2 * P8] + sq[:, 2 * P8:] + 1e-12)
    adj = (dcol < CUT) & (ii != jj)
    acol = jnp.where(adj, _C0 / dcol, 0.0)
    pencol = jnp.where(adj, 0.0, -200.0)

    ra = jax.lax.broadcasted_iota(i32, (P8, P8 * R), 0)
    rb = jax.lax.broadcasted_iota(i32, (P8, P8 * R), 1)
    REPR = (ra == rb // R).astype(f32)
    drep = jnp.dot(dcol, REPR, preferred_element_type=f32)

    kf = ((jax.lax.broadcasted_iota(i32, (1, P8 * R), 1) % R + 1)
          .astype(f32) * (math.pi / CUT))
    theta = drep * kf
    n = jnp.round(theta * (0.5 / math.pi))
    v = theta - n * (2.0 * math.pi)
    v2 = v * v
    s = v * (0.9999994441442891 + v2 * (-0.1666651950620369 + v2 * (
        0.00833220729172304 + v2 * (-0.00019803942981621122 + v2 * (
            2.694818791282763e-06 + v2 * -2.0177080094133367e-08)))))
    samp = s * jnp.dot(acol, REPR, preferred_element_type=f32)
    saug = jnp.concatenate([samp, pencol], axis=1)

    se0 = jax.lax.broadcasted_iota(i32, (M // P8, M), 0)
    se1 = jax.lax.broadcasted_iota(i32, (M // P8, M), 1)
    SELS = [(A * (se0 % GB) + P8 * (se0 // GB) + p == se1).astype(f32)
            for p in range(P8)]

    for (Wh_ref, Wr_ref, Wuh_ref, Wua_ref, bu_ref) in (
            (Wh0_ref, Wr0_ref, Wuh0_ref, Wua0_ref, bu0_ref),
            (Wh1_ref, Wr1_ref, Wuh1_ref, Wua1_ref, bu1_ref),
            (Wh2_ref, Wr2_ref, Wuh2_ref, Wua2_ref, bu2_ref)):
        q = jnp.dot(h, Wh_ref[...], preferred_element_type=f32)
        q8 = jnp.concatenate(
            [jnp.dot(S, q, preferred_element_type=f32) for S in SELS],
            axis=1)
        qt = jnp.broadcast_to(q8.reshape(AQ, GB, 1, WL),
                              (AQ, GB, A, WL)).reshape(PQ, WL)
        z2 = jnp.dot(saug, Wr_ref[...], preferred_element_type=f32)
        xm = qt + z2
        wm = xm * (0.7978845608028654 + 0.035677408136300125 * (xm * xm))
        rm = 0.5 * xm
        m = rm + rm * jnp.tanh(wm)
        mw = jnp.dot(m, Wua_ref[...], preferred_element_type=f32)
        aggw = jnp.sum(mw.reshape(AQ, M, H), axis=0)
        upd = gelu(jnp.dot(h, Wuh_ref[...], preferred_element_type=f32)
                   + aggw + bu_ref[...])
        h = h + upd

    xg = jnp.sum(h.reshape(GB, A, H), axis=1) * (1.0 / A)
    z = gelu(jnp.dot(xg, W1g_ref[...], preferred_element_type=f32)
             + jnp.dot(mol, W1m_ref[...], preferred_element_type=f32)
             + bf1_ref[...])
    z = gelu(jnp.dot(z, W2_ref[...], preferred_element_type=f32) + bf2_ref[...])
    out_ref[...] = jnp.dot(z, Wo_ref[...], preferred_element_type=f32)


def kernel(x, pos, batch, ptr, aux_ind, num_graphs, atom_emb, W_node, b_node,
           Wh0, Wr0, Wu0, bu0, Wh1, Wr1, Wu1, bu1, Wh2, Wr2, Wu2, bu2,
           W_mol, b_mol, W_fc1, b_fc1, W_fc2, b_fc2, W_out):
    f32 = jnp.float32
    T = jnp.zeros((128, H), f32).at[:NTYPES].set(
        atom_emb @ W_node[:EMB])
    Wn = W_node[EMB:]
    posr = jnp.repeat(pos, P8, axis=1)
    posc = pos.reshape(G // GB, GB, AQ, P8, 3).transpose(0, 2, 1, 4, 3).reshape(
        G // GB, AQ, GB, 3 * P8)

    def blockdiag8(W):
        Z = jnp.zeros((P8 * R + P8, WL), f32)
        for p in range(P8):
            Z = Z.at[p * R:(p + 1) * R, p * F:(p + 1) * F].set(W)
            Z = Z.at[P8 * R + p, p * F:(p + 1) * F].set(1.0)
        return Z

    row_specs = [
        pl.BlockSpec((M, NAF), lambda g: (g, 0)),
        pl.BlockSpec((M, 3 * P8), lambda g: (g, 0)),
        pl.BlockSpec((1, AQ, GB, 3 * P8), lambda g: (g, 0, 0, 0)),
    ]

    full = lambda a: pl.BlockSpec(a.shape, lambda g: tuple(0 for _ in a.shape))
    tile8 = lambda Wua: jnp.tile(Wua, (P8, 1))
    weights = [T, Wn, b_node.reshape(1, H),
               Wh0, blockdiag8(Wr0), Wu0[:H], tile8(Wu0[H:]), bu0.reshape(1, H),
               Wh1, blockdiag8(Wr1), Wu1[:H], tile8(Wu1[H:]), bu1.reshape(1, H),
               Wh2, blockdiag8(Wr2), Wu2[:H], tile8(Wu2[H:]), bu2.reshape(1, H),
               W_mol, b_mol.reshape(1, NMF),
               W_fc1[:H], W_fc1[H:], b_fc1.reshape(1, H),
               W_fc2, b_fc2.reshape(1, H), W_out]

    out = pl.pallas_call(
        _block_kernel,
        grid=(G // GB,),
        in_specs=row_specs + [full(w) for w in weights],
        out_specs=pl.BlockSpec((GB, OUT), lambda g: (g, 0)),
        out_shape=jax.ShapeDtypeStruct((G, OUT), f32),
        compiler_params=pltpu.CompilerParams(
            dimension_semantics=("parallel",)),
        interpret=_INTERPRET,
    )(x, posr, posc, *weights)
    return out


# speedup vs baseline: 1.8844x; 1.0194x over previous
import math

import jax
import jax.numpy as jnp
from jax.experimental import pallas as pl
from jax.experimental.pallas import tpu as pltpu

G = 512
A = 32
N = G * A
H = 128
F = 64
R = 12
CUT = 5.0
NAF = 13
NMF = 8
OUT = 256
NTYPES = 101
EMB = 5

GB = 32
M = GB * A
P8 = 8
AQ = A // P8
PQ = M * AQ
WL = P8 * F

_INTERPRET = False

_C0 = math.sqrt(2.0 / CUT)


def _block_kernel(x_ref, posr_ref, posc_ref, T_ref, Wn_ref, bn_ref,
                  Wh0_ref, Wr0_ref, Wuh0_ref, Wua0_ref, bu0_ref,
                  Wh1_ref, Wr1_ref, Wuh1_ref, Wua1_ref, bu1_ref,
                  Wh2_ref, Wr2_ref, Wuh2_ref, Wua2_ref, bu2_ref,
                  Wmol_ref, bmol_ref, W1g_ref, W1m_ref, bf1_ref,
                  W2_ref, bf2_ref, Wo_ref, out_ref):
    gelu = jax.nn.gelu
    f32 = jnp.float32
    i32 = jnp.int32

    xb = x_ref[...]
    poscb = posc_ref[...]

    row = jax.lax.broadcasted_iota(i32, (M, 1), 0)
    first = (row % A == 0).astype(f32)
    molx = jnp.sum((xb * first).reshape(GB, A, NAF), axis=1)
    mol = jnp.dot(molx[:, NAF - NMF:], Wmol_ref[...],
                  preferred_element_type=f32) + bmol_ref[...]

    atype = jnp.clip((xb[:, 0:1] * NTYPES).astype(i32), 0, NTYPES - 1)
    lanes = jax.lax.broadcasted_iota(i32, (M, 128), 1)
    onehot = (lanes == atype).astype(f32)
    h = gelu(jnp.dot(onehot, T_ref[...], preferred_element_type=f32)
             + jnp.dot(xb[:, 1:], Wn_ref[...], preferred_element_type=f32)
             + bn_ref[...])

    prow = jnp.broadcast_to(posr_ref[...].reshape(1, M, 3 * P8),
                            (AQ, M, 3 * P8)).reshape(PQ, 3 * P8)
    pcol = jnp.broadcast_to(poscb.reshape(AQ, GB, 1, 3 * P8),
                            (AQ, GB, A, 3 * P8)).reshape(PQ, 3 * P8)
    df = prow - pcol
    sq = df * df

    ridx = jax.lax.broadcasted_iota(i32, (PQ, 1), 0)
    jo = ridx // M
    ii = ridx % A
    jj = P8 * jo + jax.lax.broadcasted_iota(i32, (PQ, P8), 1)
    dcol = jnp.sqrt(sq[:, 0:P8] + sq[:, P8:2 * P8] + sq[:, 2 * P8:] + 1e-12)
    adj = (dcol < CUT) & (ii != jj)
    acol = jnp.where(adj, _C0 / dcol, 0.0)
    pencol = jnp.where(adj, 0.0, -200.0)

    ra = jax.lax.broadcasted_iota(i32, (P8, P8 * R), 0)
    rb = jax.lax.broadcasted_iota(i32, (P8, P8 * R), 1)
    REPR = (ra == rb // R).astype(f32)
    drep = jnp.dot(dcol, REPR, preferred_element_type=f32)

    kf = ((jax.lax.broadcasted_iota(i32, (1, P8 * R), 1) % R + 1)
          .astype(f32) * (math.pi / CUT))
    theta = drep * kf
    n = jnp.round(theta * (0.5 / math.pi))
    v = theta - n * (2.0 * math.pi)
    v2 = v * v
    s = v * (0.9999994441442891 + v2 * (-0.1666651950620369 + v2 * (
        0.00833220729172304 + v2 * (-0.00019803942981621122 + v2 * (
            2.694818791282763e-06 + v2 * -2.0177080094133367e-08)))))
    samp = s * jnp.dot(acol, REPR, preferred_element_type=f32)
    saug = jnp.concatenate([samp, pencol], axis=1)

    se0 = jax.lax.broadcasted_iota(i32, (M // P8, M), 0)
    se1 = jax.lax.broadcasted_iota(i32, (M // P8, M), 1)
    SELS = [(A * (se0 % GB) + P8 * (se0 // GB) + p == se1).astype(f32)
            for p in range(P8)]

    for (Wh_ref, Wr_ref, Wuh_ref, Wua_ref, bu_ref) in (
            (Wh0_ref, Wr0_ref, Wuh0_ref, Wua0_ref, bu0_ref),
            (Wh1_ref, Wr1_ref, Wuh1_ref, Wua1_ref, bu1_ref),
            (Wh2_ref, Wr2_ref, Wuh2_ref, Wua2_ref, bu2_ref)):
        q = jnp.dot(h, Wh_ref[...], preferred_element_type=f32)
        q8 = jnp.concatenate(
            [jnp.dot(S, q, preferred_element_type=f32) for S in SELS],
            axis=1)
        qt = jnp.broadcast_to(q8.reshape(AQ, GB, 1, WL),
                              (AQ, GB, A, WL)).reshape(PQ, WL)
        z2 = jnp.dot(saug, Wr_ref[...], preferred_element_type=f32)
        xm = qt + z2
        wm = xm * (0.7978845608028654 + 0.035677408136300125 * (xm * xm))
        rm = 0.5 * xm
        m = rm + rm * jnp.tanh(wm)
        mw = jnp.dot(m, Wua_ref[...], preferred_element_type=f32)
        aggw = jnp.sum(mw.reshape(AQ, M, H), axis=0)
        upd = gelu(jnp.dot(h, Wuh_ref[...], preferred_element_type=f32)
                   + aggw + bu_ref[...])
        h = h + upd

    xg = jnp.sum(h.reshape(GB, A, H), axis=1) * (1.0 / A)
    z = gelu(jnp.dot(xg, W1g_ref[...], preferred_element_type=f32)
             + jnp.dot(mol, W1m_ref[...], preferred_element_type=f32)
             + bf1_ref[...])
    z = gelu(jnp.dot(z, W2_ref[...], preferred_element_type=f32) + bf2_ref[...])
    out_ref[...] = jnp.dot(z, Wo_ref[...], preferred_element_type=f32)


def kernel(x, pos, batch, ptr, aux_ind, num_graphs, atom_emb, W_node, b_node,
           Wh0, Wr0, Wu0, bu0, Wh1, Wr1, Wu1, bu1, Wh2, Wr2, Wu2, bu2,
           W_mol, b_mol, W_fc1, b_fc1, W_fc2, b_fc2, W_out):
    f32 = jnp.float32
    T = jnp.zeros((128, H), f32).at[:NTYPES].set(
        atom_emb @ W_node[:EMB])
    Wn = W_node[EMB:]
    posr = jnp.repeat(pos, P8, axis=1)
    posc = pos.reshape(G // GB, GB, AQ, P8, 3).transpose(0, 2, 1, 4, 3).reshape(
        G // GB, AQ, GB, 3 * P8)

    def blockdiag8(W):
        Z = jnp.zeros((P8 * R + P8, WL), f32)
        for p in range(P8):
            Z = Z.at[p * R:(p + 1) * R, p * F:(p + 1) * F].set(W)
            Z = Z.at[P8 * R + p, p * F:(p + 1) * F].set(1.0)
        return Z

    row_specs = [
        pl.BlockSpec((M, NAF), lambda g: (g, 0)),
        pl.BlockSpec((M, 3 * P8), lambda g: (g, 0)),
        pl.BlockSpec((1, AQ, GB, 3 * P8), lambda g: (g, 0, 0, 0)),
    ]

    full = lambda a: pl.BlockSpec(a.shape, lambda g: tuple(0 for _ in a.shape))
    tile8 = lambda Wua: jnp.tile(Wua, (P8, 1))
    weights = [T, Wn, b_node.reshape(1, H),
               Wh0, blockdiag8(Wr0), Wu0[:H], tile8(Wu0[H:]), bu0.reshape(1, H),
               Wh1, blockdiag8(Wr1), Wu1[:H], tile8(Wu1[H:]), bu1.reshape(1, H),
               Wh2, blockdiag8(Wr2), Wu2[:H], tile8(Wu2[H:]), bu2.reshape(1, H),
               W_mol, b_mol.reshape(1, NMF),
               W_fc1[:H], W_fc1[H:], b_fc1.reshape(1, H),
               W_fc2, b_fc2.reshape(1, H), W_out]

    out = pl.pallas_call(
        _block_kernel,
        grid=(G // GB,),
        in_specs=row_specs + [full(w) for w in weights],
        out_specs=pl.BlockSpec((GB, OUT), lambda g: (g, 0)),
        out_shape=jax.ShapeDtypeStruct((G, OUT), f32),
        compiler_params=pltpu.CompilerParams(
            dimension_semantics=("parallel",)),
        interpret=_INTERPRET,
    )(x, posr, posc, *weights)
    return out
